# baseline (speedup 1.0000x reference)
import functools

import jax
import jax.numpy as jnp
import numpy as np
from jax import lax
from jax.experimental import pallas as pl
from jax.experimental.pallas import tpu as pltpu
from jax.experimental.pallas import tpu_sc as plsc

B = 16384
L = 50
D = 16
NR = 5
NV = 1000000
NU = 1000000
N = B * L
S = 8
PR = N // S
W128 = S * D

_NC = 2
_NS = 16
_NW = _NC * _NS
_V_PER_W = N // _NW
_U_PER_W = B // _NW
_CHUNK = 2560
_NSTEP = _V_PER_W // _CHUNK


@functools.cache
def _make_sc_gather():
    mesh = plsc.VectorSubcoreMesh(core_axis_name="c", subcore_axis_name="s",
                                  num_cores=_NC, num_subcores=_NS)

    @functools.partial(
        pl.kernel,
        out_type=[
            jax.ShapeDtypeStruct((N, D), jnp.float32),
            jax.ShapeDtypeStruct((B, D), jnp.float32),
        ],
        mesh=mesh,
        compiler_params=pltpu.CompilerParams(use_tc_tiling_on_sc=False),
        scratch_types=[
            pltpu.VMEM((_CHUNK,), jnp.int32),
            pltpu.VMEM((_CHUNK, D), jnp.float32),
            pltpu.VMEM((_U_PER_W,), jnp.int32),
            pltpu.VMEM((_U_PER_W, D), jnp.float32),
            pltpu.SemaphoreType.DMA,
        ],
    )
    def _sc_gather(v2e_hbm, vidx_hbm, u2e_hbm, nodes_hbm,
                   euv_hbm, urep_hbm,
                   idx_v, rows_v, uidx_v, urows_v, sem):
        wid = lax.axis_index("s") * _NC + lax.axis_index("c")
        v2e2 = v2e_hbm
        u2e2 = u2e_hbm

        ubase = pl.multiple_of(wid * _U_PER_W, 8)
        pltpu.sync_copy(nodes_hbm.at[pl.ds(ubase, _U_PER_W)], uidx_v)
        pltpu.async_copy(u2e2.at[uidx_v], urows_v, sem).wait()
        pltpu.sync_copy(urows_v, urep_hbm.at[pl.ds(ubase, _U_PER_W)])

        vbase = wid * _V_PER_W

        def step(i, carry):
            base = pl.multiple_of(vbase + i * _CHUNK, 8)
            pltpu.sync_copy(vidx_hbm.at[pl.ds(base, _CHUNK)], idx_v)
            pltpu.async_copy(v2e2.at[idx_v], rows_v, sem).wait()
            pltpu.sync_copy(rows_v, euv_hbm.at[pl.ds(base, _CHUNK)])
            return carry

        lax.fori_loop(0, _NSTEP, step, 0)

    return _sc_gather


_PBLK = 8000


def _pack_body(v_ref, u_ref, vo_ref, uo_ref):
    def pack(x):
        x3 = x.reshape(_PBLK // 8, 8, D)
        return jnp.concatenate([x3[:, s, :] for s in range(8)], axis=1)
    vo_ref[...] = pack(v_ref[...])
    uo_ref[...] = pack(u_ref[...])


def _pack_tables(v2e, u2e):
    grid = (NV // _PBLK,)
    return pl.pallas_call(
        _pack_body,
        grid=grid,
        in_specs=[pl.BlockSpec((_PBLK, D), lambda i: (i, 0)),
                  pl.BlockSpec((_PBLK, D), lambda i: (i, 0))],
        out_specs=[pl.BlockSpec((_PBLK // 8, 128), lambda i: (i, 0)),
                   pl.BlockSpec((_PBLK // 8, 128), lambda i: (i, 0))],
        out_shape=[jax.ShapeDtypeStruct((NV // 8, 128), jnp.float32),
                   jax.ShapeDtypeStruct((NU // 8, 128), jnp.float32)],
        compiler_params=pltpu.CompilerParams(
            dimension_semantics=("parallel",)),
    )(v2e, u2e)


_GB = 16
_BBLK = _GB * S
_MB = _GB * L


def _tc_body(euv_ref, hr40_ref, urep_ref,
             c1p_ref, w1ap_ref, w2p_ref, b2p_ref,
             a1ap_ref, a1bp_ref, ba1p_ref, a2p_ref, ba2p_ref,
             a3p_ref, r8_ref, out_ref):
    f32 = jnp.float32
    dot = functools.partial(jnp.dot, preferred_element_type=f32)

    euv = euv_ref[...]
    hr40 = hr40_ref[...]
    ohp = (hr40 == lax.broadcasted_iota(jnp.int32, (_MB, S * NR), 1) % NR
           ).astype(f32)

    x1 = jnp.maximum(dot(euv, w1ap_ref[...]) + dot(ohp, c1p_ref[...]), 0.0)
    o = jnp.maximum(dot(x1, w2p_ref[...]) + b2p_ref[...], 0.0)

    etg = (lax.broadcasted_iota(jnp.int32, (_MB, _GB), 0) // L ==
           lax.broadcasted_iota(jnp.int32, (_MB, _GB), 1)).astype(f32)
    eg = (lax.broadcasted_iota(jnp.int32, (_GB, _MB), 0) ==
          lax.broadcasted_iota(jnp.int32, (_GB, _MB), 1) // L).astype(f32)

    u_att = dot(urep_ref[...], a1bp_ref[...]) + ba1p_ref[...]
    u_exp = dot(etg, u_att)

    a1 = jnp.maximum(dot(o, a1ap_ref[...]) + u_exp, 0.0)
    a2 = jnp.maximum(dot(a1, a2p_ref[...]) + ba2p_ref[...], 0.0)
    lg = dot(a2, a3p_ref[...])

    el = jnp.exp(lg)
    den = dot(eg, el)
    dexp = dot(etg, 1.0 / den)
    att = el * dexp
    att128 = dot(att, r8_ref[...])
    out_ref[...] = dot(eg, o * att128)


def _tc_call(euv128, hr40, urep128, c1p, w1ap, w2p, b2p,
             a1ap, a1bp, ba1p, a2p, ba2p, a3p, r8):
    grid = (B // _BBLK,)
    full = lambda shape: pl.BlockSpec(shape, lambda i: (0, 0))
    return pl.pallas_call(
        _tc_body,
        grid=grid,
        in_specs=[
            pl.BlockSpec((_MB, W128), lambda i: (i, 0)),
            pl.BlockSpec((_MB, S * NR), lambda i: (i, 0)),
            pl.BlockSpec((_GB, W128), lambda i: (i, 0)),
            full((S * NR, W128)),
            full((W128, W128)), full((W128, W128)), full((1, W128)),
            full((W128, W128)), full((W128, W128)), full((1, W128)),
            full((W128, W128)), full((1, W128)),
            full((W128, S)), full((S, W128)),
        ],
        out_specs=pl.BlockSpec((_GB, W128), lambda i: (i, 0)),
        out_shape=jax.ShapeDtypeStruct((B // S, W128), jnp.float32),
        compiler_params=pltpu.CompilerParams(
            dimension_semantics=("parallel",)),
    )(euv128, hr40, urep128, c1p, w1ap, w2p, b2p,
      a1ap, a1bp, ba1p, a2p, ba2p, a3p, r8)


def kernel(nodes, history_uv, history_r, v2e, u2e, r2e,
           w_r1_W, w_r1_b, w_r2_W, w_r2_b,
           att1_W, att1_b, att2_W, att2_b, att3_W, att3_b):
    f32 = jnp.float32
    nblk = B // _BBLK

    perm4 = lambda a: a.reshape(nblk, _GB, S, L).transpose(0, 1, 3, 2)
    vidx = perm4(history_uv).reshape(N).astype(jnp.int32)
    hrp = perm4(history_r).reshape(PR, S).astype(jnp.int32)
    hr40 = jnp.repeat(hrp, NR, axis=1)

    v2e128, u2e128 = _pack_tables(v2e, u2e)
    euv, urep = _make_sc_gather()(
        v2e128.reshape(NV, D), vidx, u2e128.reshape(NU, D),
        nodes.astype(jnp.int32))
    euv128 = euv.reshape(PR, W128)
    urep128 = urep.reshape(B // S, W128)

    eye8 = jnp.eye(S, dtype=f32)
    kron = lambda w: jnp.kron(eye8, w.astype(f32))
    tile8 = lambda b: jnp.tile(b.reshape(1, -1), (1, S))

    c1 = r2e @ w_r1_W[D:] + w_r1_b
    out128 = _tc_call(
        euv128, hr40, urep128,
        kron(c1), kron(w_r1_W[:D]),
        kron(w_r2_W), tile8(w_r2_b),
        kron(att1_W[:D]), kron(att1_W[D:]), tile8(att1_b),
        kron(att2_W), tile8(att2_b),
        kron(att3_W),
        kron(jnp.ones((1, D), f32)),
    )
    return out128.reshape(B, D)


# speedup vs baseline: 1.0055x; 1.0055x over previous
import functools

import jax
import jax.numpy as jnp
import numpy as np
from jax import lax
from jax.experimental import pallas as pl
from jax.experimental.pallas import tpu as pltpu
from jax.experimental.pallas import tpu_sc as plsc

B = 16384
L = 50
D = 16
NR = 5
NV = 1000000
NU = 1000000
N = B * L
S = 8
PR = N // S
W128 = S * D

_NC = 2
_NS = 16
_NW = _NC * _NS
_V_PER_W = N // _NW
_U_PER_W = B // _NW
_CHUNK = 2560
_NSTEP = _V_PER_W // _CHUNK


@functools.cache
def _make_sc_gather():
    mesh = plsc.VectorSubcoreMesh(core_axis_name="c", subcore_axis_name="s",
                                  num_cores=_NC, num_subcores=_NS)

    @functools.partial(
        pl.kernel,
        out_type=[
            jax.ShapeDtypeStruct((N, D), jnp.float32),
            jax.ShapeDtypeStruct((B, D), jnp.float32),
        ],
        mesh=mesh,
        compiler_params=pltpu.CompilerParams(use_tc_tiling_on_sc=False),
        scratch_types=[
            pltpu.VMEM((_CHUNK,), jnp.int32),
            pltpu.VMEM((_CHUNK, D), jnp.float32),
            pltpu.VMEM((_U_PER_W,), jnp.int32),
            pltpu.VMEM((_U_PER_W, D), jnp.float32),
            pltpu.SemaphoreType.DMA,
        ],
    )
    def _sc_gather(v2e_hbm, vidx_hbm, u2e_hbm, nodes_hbm,
                   euv_hbm, urep_hbm,
                   idx_v, rows_v, uidx_v, urows_v, sem):
        wid = lax.axis_index("s") * _NC + lax.axis_index("c")
        v2e2 = v2e_hbm
        u2e2 = u2e_hbm

        ubase = pl.multiple_of(wid * _U_PER_W, 8)
        pltpu.sync_copy(nodes_hbm.at[pl.ds(ubase, _U_PER_W)], uidx_v)
        pltpu.async_copy(u2e2.at[uidx_v], urows_v, sem).wait()
        pltpu.sync_copy(urows_v, urep_hbm.at[pl.ds(ubase, _U_PER_W)])

        vbase = wid * _V_PER_W

        def step(i, carry):
            base = pl.multiple_of(vbase + i * _CHUNK, 8)
            pltpu.sync_copy(vidx_hbm.at[pl.ds(base, _CHUNK)], idx_v)
            pltpu.async_copy(v2e2.at[idx_v], rows_v, sem).wait()
            pltpu.sync_copy(rows_v, euv_hbm.at[pl.ds(base, _CHUNK)])
            return carry

        lax.fori_loop(0, _NSTEP, step, 0)

    return _sc_gather


_PBLK = 8000


def _pack_body(v_ref, u_ref, vo_ref, uo_ref):
    def pack(x):
        x3 = x.reshape(_PBLK // 8, 8, D)
        return jnp.concatenate([x3[:, s, :] for s in range(8)], axis=1)
    vo_ref[...] = pack(v_ref[...])
    uo_ref[...] = pack(u_ref[...])


def _pack_tables(v2e, u2e):
    grid = (NV // _PBLK,)
    return pl.pallas_call(
        _pack_body,
        grid=grid,
        in_specs=[pl.BlockSpec((_PBLK, D), lambda i: (i, 0)),
                  pl.BlockSpec((_PBLK, D), lambda i: (i, 0))],
        out_specs=[pl.BlockSpec((_PBLK // 8, 128), lambda i: (i, 0)),
                   pl.BlockSpec((_PBLK // 8, 128), lambda i: (i, 0))],
        out_shape=[jax.ShapeDtypeStruct((NV // 8, 128), jnp.float32),
                   jax.ShapeDtypeStruct((NU // 8, 128), jnp.float32)],
        compiler_params=pltpu.CompilerParams(
            dimension_semantics=("parallel",)),
    )(v2e, u2e)


_IGB = 128


def _permute_body(uv_ref, r_ref, uvo_ref, ro_ref):
    uvo_ref[...] = jnp.transpose(uv_ref[...], (0, 2, 1))
    ro_ref[...] = jnp.transpose(r_ref[...], (0, 2, 1))


def _permute_idx(history_uv3, history_r3):
    grid = (B // S // _IGB,)
    spec_in = pl.BlockSpec((_IGB, S, L), lambda i: (i, 0, 0))
    spec_out = pl.BlockSpec((_IGB, L, S), lambda i: (i, 0, 0))
    return pl.pallas_call(
        _permute_body,
        grid=grid,
        in_specs=[spec_in, spec_in],
        out_specs=[spec_out, spec_out],
        out_shape=[jax.ShapeDtypeStruct((B // S, L, S), jnp.int32),
                   jax.ShapeDtypeStruct((B // S, L, S), jnp.int32)],
        compiler_params=pltpu.CompilerParams(
            dimension_semantics=("parallel",)),
    )(history_uv3, history_r3)


_GB = 16
_BBLK = _GB * S
_MB = _GB * L


def _tc_body(euv_ref, hrp_ref, urep_ref,
             c1p_ref, w1ap_ref, w2p_ref, b2p_ref,
             a1ap_ref, a1bp_ref, ba1p_ref, a2p_ref, ba2p_ref,
             a3p_ref, r8_ref, r5_ref, out_ref):
    f32 = jnp.float32
    dot = functools.partial(jnp.dot, preferred_element_type=f32)

    euv = euv_ref[...]
    hr40 = dot(hrp_ref[...].astype(f32), r5_ref[...])
    ohp = (hr40 == (lax.broadcasted_iota(jnp.int32, (_MB, S * NR), 1) % NR
                    ).astype(f32)).astype(f32)

    x1 = jnp.maximum(dot(euv, w1ap_ref[...]) + dot(ohp, c1p_ref[...]), 0.0)
    o = jnp.maximum(dot(x1, w2p_ref[...]) + b2p_ref[...], 0.0)

    etg = (lax.broadcasted_iota(jnp.int32, (_MB, _GB), 0) // L ==
           lax.broadcasted_iota(jnp.int32, (_MB, _GB), 1)).astype(f32)
    eg = (lax.broadcasted_iota(jnp.int32, (_GB, _MB), 0) ==
          lax.broadcasted_iota(jnp.int32, (_GB, _MB), 1) // L).astype(f32)

    u_att = dot(urep_ref[...], a1bp_ref[...]) + ba1p_ref[...]
    u_exp = dot(etg, u_att)

    a1 = jnp.maximum(dot(o, a1ap_ref[...]) + u_exp, 0.0)
    a2 = jnp.maximum(dot(a1, a2p_ref[...]) + ba2p_ref[...], 0.0)
    lg = dot(a2, a3p_ref[...])

    el = jnp.exp(lg)
    den = dot(eg, el)
    dexp = dot(etg, 1.0 / den)
    att = el * dexp
    att128 = dot(att, r8_ref[...])
    out_ref[...] = dot(eg, o * att128)


def _tc_call(euv128, hrp, urep128, c1p, w1ap, w2p, b2p,
             a1ap, a1bp, ba1p, a2p, ba2p, a3p, r8, r5):
    grid = (B // _BBLK,)
    full = lambda shape: pl.BlockSpec(shape, lambda i: (0, 0))
    return pl.pallas_call(
        _tc_body,
        grid=grid,
        in_specs=[
            pl.BlockSpec((_MB, W128), lambda i: (i, 0)),
            pl.BlockSpec((_MB, S), lambda i: (i, 0)),
            pl.BlockSpec((_GB, W128), lambda i: (i, 0)),
            full((S * NR, W128)),
            full((W128, W128)), full((W128, W128)), full((1, W128)),
            full((W128, W128)), full((W128, W128)), full((1, W128)),
            full((W128, W128)), full((1, W128)),
            full((W128, S)), full((S, W128)), full((S, S * NR)),
        ],
        out_specs=pl.BlockSpec((_GB, W128), lambda i: (i, 0)),
        out_shape=jax.ShapeDtypeStruct((B // S, W128), jnp.float32),
        compiler_params=pltpu.CompilerParams(
            dimension_semantics=("parallel",)),
    )(euv128, hrp, urep128, c1p, w1ap, w2p, b2p,
      a1ap, a1bp, ba1p, a2p, ba2p, a3p, r8, r5)


def kernel(nodes, history_uv, history_r, v2e, u2e, r2e,
           w_r1_W, w_r1_b, w_r2_W, w_r2_b,
           att1_W, att1_b, att2_W, att2_b, att3_W, att3_b):
    f32 = jnp.float32

    i32 = jnp.int32
    uv3, r3 = _permute_idx(history_uv.astype(i32).reshape(B // S, S, L),
                           history_r.astype(i32).reshape(B // S, S, L))
    vidx = uv3.reshape(N)
    hrp = r3.reshape(PR, S)

    v2e128, u2e128 = _pack_tables(v2e, u2e)
    euv, urep = _make_sc_gather()(
        v2e128.reshape(NV, D), vidx, u2e128.reshape(NU, D),
        nodes.astype(jnp.int32))
    euv128 = euv.reshape(PR, W128)
    urep128 = urep.reshape(B // S, W128)

    eye8 = jnp.eye(S, dtype=f32)
    kron = lambda w: jnp.kron(eye8, w.astype(f32))
    tile8 = lambda b: jnp.tile(b.reshape(1, -1), (1, S))

    c1 = r2e @ w_r1_W[D:] + w_r1_b
    out128 = _tc_call(
        euv128, hrp, urep128,
        kron(c1), kron(w_r1_W[:D]),
        kron(w_r2_W), tile8(w_r2_b),
        kron(att1_W[:D]), kron(att1_W[D:]), tile8(att1_b),
        kron(att2_W), tile8(att2_b),
        kron(att3_W),
        kron(jnp.ones((1, D), f32)),
        kron(jnp.ones((1, NR), f32)),
    )
    return out128.reshape(B, D)


# speedup vs baseline: 1.0161x; 1.0105x over previous
import functools

import jax
import jax.numpy as jnp
import numpy as np
from jax import lax
from jax.experimental import pallas as pl
from jax.experimental.pallas import tpu as pltpu
from jax.experimental.pallas import tpu_sc as plsc

B = 16384
L = 50
D = 16
NR = 5
NV = 1000000
NU = 1000000
N = B * L
S = 8
PR = N // S
W128 = S * D

_NC = 2
_NS = 16
_NW = _NC * _NS
_V_PER_W = N // _NW
_U_PER_W = B // _NW
_CHUNK = 2560
_NSTEP = _V_PER_W // _CHUNK


@functools.cache
def _make_sc_gather():
    mesh = plsc.VectorSubcoreMesh(core_axis_name="c", subcore_axis_name="s",
                                  num_cores=_NC, num_subcores=_NS)

    @functools.partial(
        pl.kernel,
        out_type=[
            jax.ShapeDtypeStruct((N, D), jnp.float32),
            jax.ShapeDtypeStruct((B, D), jnp.float32),
        ],
        mesh=mesh,
        compiler_params=pltpu.CompilerParams(use_tc_tiling_on_sc=False),
        scratch_types=[
            pltpu.VMEM((_CHUNK,), jnp.int32),
            pltpu.VMEM((_CHUNK, D), jnp.float32),
            pltpu.VMEM((_U_PER_W,), jnp.int32),
            pltpu.VMEM((_U_PER_W, D), jnp.float32),
            pltpu.SemaphoreType.DMA,
        ],
    )
    def _sc_gather(v2e_hbm, vidx_hbm, u2e_hbm, nodes_hbm,
                   euv_hbm, urep_hbm,
                   idx_v, rows_v, uidx_v, urows_v, sem):
        wid = lax.axis_index("s") * _NC + lax.axis_index("c")
        v2e2 = v2e_hbm
        u2e2 = u2e_hbm

        ubase = pl.multiple_of(wid * _U_PER_W, 8)
        pltpu.sync_copy(nodes_hbm.at[pl.ds(ubase, _U_PER_W)], uidx_v)
        pltpu.async_copy(u2e2.at[uidx_v], urows_v, sem).wait()
        pltpu.sync_copy(urows_v, urep_hbm.at[pl.ds(ubase, _U_PER_W)])

        vbase = wid * _V_PER_W

        def step(i, carry):
            base = pl.multiple_of(vbase + i * _CHUNK, 8)
            pltpu.sync_copy(vidx_hbm.at[pl.ds(base, _CHUNK)], idx_v)
            pltpu.async_copy(v2e2.at[idx_v], rows_v, sem).wait()
            pltpu.sync_copy(rows_v, euv_hbm.at[pl.ds(base, _CHUNK)])
            return carry

        lax.fori_loop(0, _NSTEP, step, 0)

    return _sc_gather


_PBLK = 8000


def _pack_body(v_ref, u_ref, vo_ref, uo_ref):
    def pack(x):
        x3 = x.reshape(_PBLK // 8, 8, D)
        return jnp.concatenate([x3[:, s, :] for s in range(8)], axis=1)
    vo_ref[...] = pack(v_ref[...])
    uo_ref[...] = pack(u_ref[...])


def _pack_tables(v2e, u2e):
    grid = (NV // _PBLK,)
    return pl.pallas_call(
        _pack_body,
        grid=grid,
        in_specs=[pl.BlockSpec((_PBLK, D), lambda i: (i, 0)),
                  pl.BlockSpec((_PBLK, D), lambda i: (i, 0))],
        out_specs=[pl.BlockSpec((_PBLK // 8, 128), lambda i: (i, 0)),
                   pl.BlockSpec((_PBLK // 8, 128), lambda i: (i, 0))],
        out_shape=[jax.ShapeDtypeStruct((NV // 8, 128), jnp.float32),
                   jax.ShapeDtypeStruct((NU // 8, 128), jnp.float32)],
        compiler_params=pltpu.CompilerParams(
            dimension_semantics=("parallel",)),
    )(v2e, u2e)


_ICB = 1024


def _permute_body(uv_ref, r_ref, uvo_ref, ro_ref):
    def perm(x):
        t = jnp.transpose(x)
        return jnp.transpose(t.reshape(_ICB // S, S, L), (0, 2, 1))
    uvo_ref[...] = perm(uv_ref[...])
    ro_ref[...] = perm(r_ref[...])


def _permute_idx(hu_t, hr_t):
    grid = (B // _ICB,)
    spec_in = pl.BlockSpec((L, _ICB), lambda i: (0, i))
    spec_out = pl.BlockSpec((_ICB // S, L, S), lambda i: (i, 0, 0))
    return pl.pallas_call(
        _permute_body,
        grid=grid,
        in_specs=[spec_in, spec_in],
        out_specs=[spec_out, spec_out],
        out_shape=[jax.ShapeDtypeStruct((B // S, L, S), jnp.int32),
                   jax.ShapeDtypeStruct((B // S, L, S), jnp.int32)],
        compiler_params=pltpu.CompilerParams(
            dimension_semantics=("parallel",)),
    )(hu_t, hr_t)


_GB = 16
_BBLK = _GB * S
_MB = _GB * L


def _tc_body(euv_ref, hrp_ref, urep_ref,
             c1p_ref, w1ap_ref, w2p_ref, b2p_ref,
             a1ap_ref, a1bp_ref, ba1p_ref, a2p_ref, ba2p_ref,
             a3p_ref, r8_ref, r5_ref, out_ref):
    f32 = jnp.float32
    dot = functools.partial(jnp.dot, preferred_element_type=f32)

    euv = euv_ref[...]
    hr40 = dot(hrp_ref[...].astype(f32), r5_ref[...])
    ohp = (hr40 == (lax.broadcasted_iota(jnp.int32, (_MB, S * NR), 1) % NR
                    ).astype(f32)).astype(f32)

    x1 = jnp.maximum(dot(euv, w1ap_ref[...]) + dot(ohp, c1p_ref[...]), 0.0)
    o = jnp.maximum(dot(x1, w2p_ref[...]) + b2p_ref[...], 0.0)

    etg = (lax.broadcasted_iota(jnp.int32, (_MB, _GB), 0) // L ==
           lax.broadcasted_iota(jnp.int32, (_MB, _GB), 1)).astype(f32)
    eg = (lax.broadcasted_iota(jnp.int32, (_GB, _MB), 0) ==
          lax.broadcasted_iota(jnp.int32, (_GB, _MB), 1) // L).astype(f32)

    u_att = dot(urep_ref[...], a1bp_ref[...]) + ba1p_ref[...]
    u_exp = dot(etg, u_att)

    a1 = jnp.maximum(dot(o, a1ap_ref[...]) + u_exp, 0.0)
    a2 = jnp.maximum(dot(a1, a2p_ref[...]) + ba2p_ref[...], 0.0)
    lg = dot(a2, a3p_ref[...])

    el = jnp.exp(lg)
    den = dot(eg, el)
    dexp = dot(etg, 1.0 / den)
    att = el * dexp
    att128 = dot(att, r8_ref[...])
    out_ref[...] = dot(eg, o * att128)


def _tc_call(euv128, hrp, urep128, c1p, w1ap, w2p, b2p,
             a1ap, a1bp, ba1p, a2p, ba2p, a3p, r8, r5):
    grid = (B // _BBLK,)
    full = lambda shape: pl.BlockSpec(shape, lambda i: (0, 0))
    return pl.pallas_call(
        _tc_body,
        grid=grid,
        in_specs=[
            pl.BlockSpec((_MB, W128), lambda i: (i, 0)),
            pl.BlockSpec((_MB, S), lambda i: (i, 0)),
            pl.BlockSpec((_GB, W128), lambda i: (i, 0)),
            full((S * NR, W128)),
            full((W128, W128)), full((W128, W128)), full((1, W128)),
            full((W128, W128)), full((W128, W128)), full((1, W128)),
            full((W128, W128)), full((1, W128)),
            full((W128, S)), full((S, W128)), full((S, S * NR)),
        ],
        out_specs=pl.BlockSpec((_GB, W128), lambda i: (i, 0)),
        out_shape=jax.ShapeDtypeStruct((B // S, W128), jnp.float32),
        compiler_params=pltpu.CompilerParams(
            dimension_semantics=("parallel",)),
    )(euv128, hrp, urep128, c1p, w1ap, w2p, b2p,
      a1ap, a1bp, ba1p, a2p, ba2p, a3p, r8, r5)


def kernel(nodes, history_uv, history_r, v2e, u2e, r2e,
           w_r1_W, w_r1_b, w_r2_W, w_r2_b,
           att1_W, att1_b, att2_W, att2_b, att3_W, att3_b):
    f32 = jnp.float32

    i32 = jnp.int32
    uv3, r3 = _permute_idx(history_uv.astype(i32).T, history_r.astype(i32).T)
    vidx = uv3.reshape(N)
    hrp = r3.reshape(PR, S)

    v2e128, u2e128 = _pack_tables(v2e, u2e)
    euv, urep = _make_sc_gather()(
        v2e128.reshape(NV, D), vidx, u2e128.reshape(NU, D),
        nodes.astype(jnp.int32))
    euv128 = euv.reshape(PR, W128)
    urep128 = urep.reshape(B // S, W128)

    eye8 = jnp.eye(S, dtype=f32)
    kron = lambda w: jnp.kron(eye8, w.astype(f32))
    tile8 = lambda b: jnp.tile(b.reshape(1, -1), (1, S))

    c1 = r2e @ w_r1_W[D:] + w_r1_b
    out128 = _tc_call(
        euv128, hrp, urep128,
        kron(c1), kron(w_r1_W[:D]),
        kron(w_r2_W), tile8(w_r2_b),
        kron(att1_W[:D]), kron(att1_W[D:]), tile8(att1_b),
        kron(att2_W), tile8(att2_b),
        kron(att3_W),
        kron(jnp.ones((1, D), f32)),
        kron(jnp.ones((1, NR), f32)),
    )
    return out128.reshape(B, D)


# speedup vs baseline: 1.4573x; 1.4342x over previous
import functools

import jax
import jax.numpy as jnp
import numpy as np
from jax import lax
from jax.experimental import pallas as pl
from jax.experimental.pallas import tpu as pltpu
from jax.experimental.pallas import tpu_sc as plsc

B = 16384
L = 50
D = 16
NR = 5
NV = 1000000
NU = 1000000
N = B * L
S = 8
PR = N // S
W128 = S * D

_NC = 2
_NS = 16
_NW = _NC * _NS
_V_PER_W = N // _NW
_U_PER_W = B // _NW
_CHUNK = 2560
_NSTEP = _V_PER_W // _CHUNK


@functools.cache
def _make_sc_gather():
    mesh = plsc.VectorSubcoreMesh(core_axis_name="c", subcore_axis_name="s",
                                  num_cores=_NC, num_subcores=_NS)

    @functools.partial(
        pl.kernel,
        out_type=[
            jax.ShapeDtypeStruct((N, D), jnp.float32),
            jax.ShapeDtypeStruct((B, D), jnp.float32),
        ],
        mesh=mesh,
        compiler_params=pltpu.CompilerParams(use_tc_tiling_on_sc=False),
        scratch_types=[
            pltpu.VMEM((_CHUNK,), jnp.int32),
            pltpu.VMEM((_CHUNK, D), jnp.float32),
            pltpu.VMEM((_U_PER_W,), jnp.int32),
            pltpu.VMEM((_U_PER_W, D), jnp.float32),
            pltpu.SemaphoreType.DMA,
        ],
    )
    def _sc_gather(v2e_hbm, vidx_hbm, u2e_hbm, nodes_hbm,
                   euv_hbm, urep_hbm,
                   idx_v, rows_v, uidx_v, urows_v, sem):
        wid = lax.axis_index("s") * _NC + lax.axis_index("c")
        v2e2 = v2e_hbm
        u2e2 = u2e_hbm

        ubase = pl.multiple_of(wid * _U_PER_W, 8)
        pltpu.sync_copy(nodes_hbm.at[pl.ds(ubase, _U_PER_W)], uidx_v)
        pltpu.async_copy(u2e2.at[uidx_v], urows_v, sem).wait()
        pltpu.sync_copy(urows_v, urep_hbm.at[pl.ds(ubase, _U_PER_W)])

        vbase = wid * _V_PER_W

        def step(i, carry):
            base = pl.multiple_of(vbase + i * _CHUNK, 8)
            pltpu.sync_copy(vidx_hbm.at[pl.ds(base, _CHUNK)], idx_v)
            pltpu.async_copy(v2e2.at[idx_v], rows_v, sem).wait()
            pltpu.sync_copy(rows_v, euv_hbm.at[pl.ds(base, _CHUNK)])
            return carry

        lax.fori_loop(0, _NSTEP, step, 0)

    return _sc_gather


_PBLK = 8192


def _pack_body(vt_ref, ut_ref, vo_ref, uo_ref):
    def pack(xt):
        x3 = jnp.transpose(xt).reshape(_PBLK // 8, 8, D)
        return jnp.concatenate([x3[:, s, :] for s in range(8)], axis=1)
    vo_ref[...] = pack(vt_ref[...])
    uo_ref[...] = pack(ut_ref[...])


def _pack_tables(v2e_t, u2e_t):
    grid = (pl.cdiv(NV, _PBLK),)
    return pl.pallas_call(
        _pack_body,
        grid=grid,
        in_specs=[pl.BlockSpec((D, _PBLK), lambda i: (0, i)),
                  pl.BlockSpec((D, _PBLK), lambda i: (0, i))],
        out_specs=[pl.BlockSpec((_PBLK // 8, 128), lambda i: (i, 0)),
                   pl.BlockSpec((_PBLK // 8, 128), lambda i: (i, 0))],
        out_shape=[jax.ShapeDtypeStruct((NV // 8, 128), jnp.float32),
                   jax.ShapeDtypeStruct((NU // 8, 128), jnp.float32)],
        compiler_params=pltpu.CompilerParams(
            dimension_semantics=("parallel",)),
    )(v2e_t, u2e_t)


_ICB = 1024


def _permute_body(uv_ref, r_ref, uvo_ref, ro_ref):
    def perm(x):
        t = jnp.transpose(x)
        return jnp.transpose(t.reshape(_ICB // S, S, L), (0, 2, 1))
    uvo_ref[...] = perm(uv_ref[...])
    ro_ref[...] = perm(r_ref[...])


def _permute_idx(hu_t, hr_t):
    grid = (B // _ICB,)
    spec_in = pl.BlockSpec((L, _ICB), lambda i: (0, i))
    spec_out = pl.BlockSpec((_ICB // S, L, S), lambda i: (i, 0, 0))
    return pl.pallas_call(
        _permute_body,
        grid=grid,
        in_specs=[spec_in, spec_in],
        out_specs=[spec_out, spec_out],
        out_shape=[jax.ShapeDtypeStruct((B // S, L, S), jnp.int32),
                   jax.ShapeDtypeStruct((B // S, L, S), jnp.int32)],
        compiler_params=pltpu.CompilerParams(
            dimension_semantics=("parallel",)),
    )(hu_t, hr_t)


_GB = 16
_BBLK = _GB * S
_MB = _GB * L


def _tc_body(euv_ref, hrp_ref, urep_ref,
             c1p_ref, w1ap_ref, w2p_ref, b2p_ref,
             a1ap_ref, a1bp_ref, ba1p_ref, a2p_ref, ba2p_ref,
             a3p_ref, r8_ref, r5_ref, out_ref):
    f32 = jnp.float32
    dot = functools.partial(jnp.dot, preferred_element_type=f32)

    euv = euv_ref[...]
    hr40 = dot(hrp_ref[...].astype(f32), r5_ref[...])
    ohp = (hr40 == (lax.broadcasted_iota(jnp.int32, (_MB, S * NR), 1) % NR
                    ).astype(f32)).astype(f32)

    x1 = jnp.maximum(dot(euv, w1ap_ref[...]) + dot(ohp, c1p_ref[...]), 0.0)
    o = jnp.maximum(dot(x1, w2p_ref[...]) + b2p_ref[...], 0.0)

    etg = (lax.broadcasted_iota(jnp.int32, (_MB, _GB), 0) // L ==
           lax.broadcasted_iota(jnp.int32, (_MB, _GB), 1)).astype(f32)
    eg = (lax.broadcasted_iota(jnp.int32, (_GB, _MB), 0) ==
          lax.broadcasted_iota(jnp.int32, (_GB, _MB), 1) // L).astype(f32)

    u_att = dot(urep_ref[...], a1bp_ref[...]) + ba1p_ref[...]
    u_exp = dot(etg, u_att)

    a1 = jnp.maximum(dot(o, a1ap_ref[...]) + u_exp, 0.0)
    a2 = jnp.maximum(dot(a1, a2p_ref[...]) + ba2p_ref[...], 0.0)
    lg = dot(a2, a3p_ref[...])

    el = jnp.exp(lg)
    den = dot(eg, el)
    dexp = dot(etg, 1.0 / den)
    att = el * dexp
    att128 = dot(att, r8_ref[...])
    out_ref[...] = dot(eg, o * att128)


def _tc_call(euv128, hrp, urep128, c1p, w1ap, w2p, b2p,
             a1ap, a1bp, ba1p, a2p, ba2p, a3p, r8, r5):
    grid = (B // _BBLK,)
    full = lambda shape: pl.BlockSpec(shape, lambda i: (0, 0))
    return pl.pallas_call(
        _tc_body,
        grid=grid,
        in_specs=[
            pl.BlockSpec((_MB, W128), lambda i: (i, 0)),
            pl.BlockSpec((_MB, S), lambda i: (i, 0)),
            pl.BlockSpec((_GB, W128), lambda i: (i, 0)),
            full((S * NR, W128)),
            full((W128, W128)), full((W128, W128)), full((1, W128)),
            full((W128, W128)), full((W128, W128)), full((1, W128)),
            full((W128, W128)), full((1, W128)),
            full((W128, S)), full((S, W128)), full((S, S * NR)),
        ],
        out_specs=pl.BlockSpec((_GB, W128), lambda i: (i, 0)),
        out_shape=jax.ShapeDtypeStruct((B // S, W128), jnp.float32),
        compiler_params=pltpu.CompilerParams(
            dimension_semantics=("parallel",)),
    )(euv128, hrp, urep128, c1p, w1ap, w2p, b2p,
      a1ap, a1bp, ba1p, a2p, ba2p, a3p, r8, r5)


def kernel(nodes, history_uv, history_r, v2e, u2e, r2e,
           w_r1_W, w_r1_b, w_r2_W, w_r2_b,
           att1_W, att1_b, att2_W, att2_b, att3_W, att3_b):
    f32 = jnp.float32

    i32 = jnp.int32
    uv3, r3 = _permute_idx(history_uv.astype(i32).T, history_r.astype(i32).T)
    vidx = uv3.reshape(N)
    hrp = r3.reshape(PR, S)

    v2e128, u2e128 = _pack_tables(v2e.T, u2e.T)
    euv, urep = _make_sc_gather()(
        v2e128.reshape(NV, D), vidx, u2e128.reshape(NU, D),
        nodes.astype(jnp.int32))
    euv128 = euv.reshape(PR, W128)
    urep128 = urep.reshape(B // S, W128)

    eye8 = jnp.eye(S, dtype=f32)
    kron = lambda w: jnp.kron(eye8, w.astype(f32))
    tile8 = lambda b: jnp.tile(b.reshape(1, -1), (1, S))

    c1 = r2e @ w_r1_W[D:] + w_r1_b
    out128 = _tc_call(
        euv128, hrp, urep128,
        kron(c1), kron(w_r1_W[:D]),
        kron(w_r2_W), tile8(w_r2_b),
        kron(att1_W[:D]), kron(att1_W[D:]), tile8(att1_b),
        kron(att2_W), tile8(att2_b),
        kron(att3_W),
        kron(jnp.ones((1, D), f32)),
        kron(jnp.ones((1, NR), f32)),
    )
    return out128.reshape(B, D)


# speedup vs baseline: 2.0503x; 1.4070x over previous
import functools

import jax
import jax.numpy as jnp
import numpy as np
from jax import lax
from jax.experimental import pallas as pl
from jax.experimental.pallas import tpu as pltpu
from jax.experimental.pallas import tpu_sc as plsc

B = 16384
L = 50
D = 16
NR = 5
NV = 1000000
NU = 1000000
N = B * L
S = 8
PR = N // S
W128 = S * D

_NC = 2
_NS = 16
_NW = _NC * _NS
_V_PER_W = N // _NW
_U_PER_W = B // _NW
_CHUNK = 2560
_NSTEP = _V_PER_W // _CHUNK


@functools.cache
def _make_sc_gather():
    mesh = plsc.VectorSubcoreMesh(core_axis_name="c", subcore_axis_name="s",
                                  num_cores=_NC, num_subcores=_NS)

    @functools.partial(
        pl.kernel,
        out_type=[
            jax.ShapeDtypeStruct((N, D), jnp.float32),
            jax.ShapeDtypeStruct((B, D), jnp.float32),
        ],
        mesh=mesh,
        compiler_params=pltpu.CompilerParams(use_tc_tiling_on_sc=False),
        scratch_types=[
            pltpu.VMEM((_CHUNK,), jnp.int32),
            pltpu.VMEM((_CHUNK, D), jnp.float32),
            pltpu.VMEM((_U_PER_W,), jnp.int32),
            pltpu.VMEM((_U_PER_W, D), jnp.float32),
            pltpu.SemaphoreType.DMA,
        ],
    )
    def _sc_gather(v2e_hbm, vidx_hbm, u2e_hbm, nodes_hbm,
                   euv_hbm, urep_hbm,
                   idx_v, rows_v, uidx_v, urows_v, sem):
        wid = lax.axis_index("s") * _NC + lax.axis_index("c")
        v2e2 = v2e_hbm
        u2e2 = u2e_hbm

        ubase = pl.multiple_of(wid * _U_PER_W, 8)
        pltpu.sync_copy(nodes_hbm.at[pl.ds(ubase, _U_PER_W)], uidx_v)
        pltpu.async_copy(u2e2.at[uidx_v], urows_v, sem).wait()
        pltpu.sync_copy(urows_v, urep_hbm.at[pl.ds(ubase, _U_PER_W)])

        vbase = wid * _V_PER_W

        def step(i, carry):
            base = pl.multiple_of(vbase + i * _CHUNK, 8)
            pltpu.sync_copy(vidx_hbm.at[pl.ds(base, _CHUNK)], idx_v)
            pltpu.async_copy(v2e2.at[idx_v], rows_v, sem).wait()
            pltpu.sync_copy(rows_v, euv_hbm.at[pl.ds(base, _CHUNK)])
            return carry

        lax.fori_loop(0, _NSTEP, step, 0)

    return _sc_gather


_PBLK = 8192
_NPB = (NV + _PBLK - 1) // _PBLK
_NVP = _NPB * _PBLK


def _sigma(i):
    return ((i >> 13) << 13) | ((i & 1023) << 3) | ((i >> 10) & 7)


def _pack_body(vt_ref, ut_ref, vo_ref, uo_ref):
    f32 = jnp.float32
    dot = functools.partial(jnp.dot, preferred_element_type=f32)
    pb8 = _PBLK // 8

    base = pl.program_id(0) * _PBLK

    def pack(xt):
        col = base + lax.broadcasted_iota(jnp.int32, (D, _PBLK), 1)
        xt = jnp.where(col < NV, xt, 0.0)
        acc = None
        for s in range(8):
            t = jnp.transpose(xt[:, s * pb8:(s + 1) * pb8])
            sh = (lax.broadcasted_iota(jnp.int32, (D, W128), 1) ==
                  lax.broadcasted_iota(jnp.int32, (D, W128), 0) + s * D
                  ).astype(f32)
            y = dot(t, sh)
            acc = y if acc is None else acc + y
        return acc

    vo_ref[...] = pack(vt_ref[...])
    uo_ref[...] = pack(ut_ref[...])


def _pack_tables(v2e_t, u2e_t):
    grid = (_NPB,)
    return pl.pallas_call(
        _pack_body,
        grid=grid,
        in_specs=[pl.BlockSpec((D, _PBLK), lambda i: (0, i)),
                  pl.BlockSpec((D, _PBLK), lambda i: (0, i))],
        out_specs=[pl.BlockSpec((_PBLK // 8, 128), lambda i: (i, 0)),
                   pl.BlockSpec((_PBLK // 8, 128), lambda i: (i, 0))],
        out_shape=[jax.ShapeDtypeStruct((_NVP // 8, 128), jnp.float32),
                   jax.ShapeDtypeStruct((_NVP // 8, 128), jnp.float32)],
        compiler_params=pltpu.CompilerParams(
            dimension_semantics=("parallel",)),
    )(v2e_t, u2e_t)


_ICB = 1024


def _permute_body(uv_ref, r_ref, uvo_ref, ro_ref):
    def perm(x):
        t = jnp.transpose(x)
        return jnp.transpose(t.reshape(_ICB // S, S, L), (0, 2, 1))
    uvo_ref[...] = _sigma(perm(uv_ref[...]))
    ro_ref[...] = perm(r_ref[...])


def _permute_idx(hu_t, hr_t):
    grid = (B // _ICB,)
    spec_in = pl.BlockSpec((L, _ICB), lambda i: (0, i))
    spec_out = pl.BlockSpec((_ICB // S, L, S), lambda i: (i, 0, 0))
    return pl.pallas_call(
        _permute_body,
        grid=grid,
        in_specs=[spec_in, spec_in],
        out_specs=[spec_out, spec_out],
        out_shape=[jax.ShapeDtypeStruct((B // S, L, S), jnp.int32),
                   jax.ShapeDtypeStruct((B // S, L, S), jnp.int32)],
        compiler_params=pltpu.CompilerParams(
            dimension_semantics=("parallel",)),
    )(hu_t, hr_t)


_GB = 16
_BBLK = _GB * S
_MB = _GB * L


def _tc_body(euv_ref, hrp_ref, urep_ref,
             c1p_ref, w1ap_ref, w2p_ref, b2p_ref,
             a1ap_ref, a1bp_ref, ba1p_ref, a2p_ref, ba2p_ref,
             a3p_ref, r8_ref, r5_ref, out_ref):
    f32 = jnp.float32
    dot = functools.partial(jnp.dot, preferred_element_type=f32)

    euv = euv_ref[...]
    hr40 = dot(hrp_ref[...].astype(f32), r5_ref[...])
    ohp = (hr40 == (lax.broadcasted_iota(jnp.int32, (_MB, S * NR), 1) % NR
                    ).astype(f32)).astype(f32)

    x1 = jnp.maximum(dot(euv, w1ap_ref[...]) + dot(ohp, c1p_ref[...]), 0.0)
    o = jnp.maximum(dot(x1, w2p_ref[...]) + b2p_ref[...], 0.0)

    etg = (lax.broadcasted_iota(jnp.int32, (_MB, _GB), 0) // L ==
           lax.broadcasted_iota(jnp.int32, (_MB, _GB), 1)).astype(f32)
    eg = (lax.broadcasted_iota(jnp.int32, (_GB, _MB), 0) ==
          lax.broadcasted_iota(jnp.int32, (_GB, _MB), 1) // L).astype(f32)

    u_att = dot(urep_ref[...], a1bp_ref[...]) + ba1p_ref[...]
    u_exp = dot(etg, u_att)

    a1 = jnp.maximum(dot(o, a1ap_ref[...]) + u_exp, 0.0)
    a2 = jnp.maximum(dot(a1, a2p_ref[...]) + ba2p_ref[...], 0.0)
    lg = dot(a2, a3p_ref[...])

    el = jnp.exp(lg)
    den = dot(eg, el)
    dexp = dot(etg, 1.0 / den)
    att = el * dexp
    att128 = dot(att, r8_ref[...])
    out_ref[...] = dot(eg, o * att128)


def _tc_call(euv128, hrp, urep128, c1p, w1ap, w2p, b2p,
             a1ap, a1bp, ba1p, a2p, ba2p, a3p, r8, r5):
    grid = (B // _BBLK,)
    full = lambda shape: pl.BlockSpec(shape, lambda i: (0, 0))
    return pl.pallas_call(
        _tc_body,
        grid=grid,
        in_specs=[
            pl.BlockSpec((_MB, W128), lambda i: (i, 0)),
            pl.BlockSpec((_MB, S), lambda i: (i, 0)),
            pl.BlockSpec((_GB, W128), lambda i: (i, 0)),
            full((S * NR, W128)),
            full((W128, W128)), full((W128, W128)), full((1, W128)),
            full((W128, W128)), full((W128, W128)), full((1, W128)),
            full((W128, W128)), full((1, W128)),
            full((W128, S)), full((S, W128)), full((S, S * NR)),
        ],
        out_specs=pl.BlockSpec((_GB, W128), lambda i: (i, 0)),
        out_shape=jax.ShapeDtypeStruct((B // S, W128), jnp.float32),
        compiler_params=pltpu.CompilerParams(
            dimension_semantics=("parallel",)),
    )(euv128, hrp, urep128, c1p, w1ap, w2p, b2p,
      a1ap, a1bp, ba1p, a2p, ba2p, a3p, r8, r5)


def kernel(nodes, history_uv, history_r, v2e, u2e, r2e,
           w_r1_W, w_r1_b, w_r2_W, w_r2_b,
           att1_W, att1_b, att2_W, att2_b, att3_W, att3_b):
    f32 = jnp.float32

    i32 = jnp.int32
    uv3, r3 = _permute_idx(history_uv.astype(i32).T, history_r.astype(i32).T)
    vidx = uv3.reshape(N)
    hrp = r3.reshape(PR, S)

    v2e128, u2e128 = _pack_tables(v2e.T, u2e.T)
    euv, urep = _make_sc_gather()(
        v2e128.reshape(_NVP, D), vidx, u2e128.reshape(_NVP, D),
        _sigma(nodes.astype(jnp.int32)))
    euv128 = euv.reshape(PR, W128)
    urep128 = urep.reshape(B // S, W128)

    eye8 = jnp.eye(S, dtype=f32)
    kron = lambda w: jnp.kron(eye8, w.astype(f32))
    tile8 = lambda b: jnp.tile(b.reshape(1, -1), (1, S))

    c1 = r2e @ w_r1_W[D:] + w_r1_b
    out128 = _tc_call(
        euv128, hrp, urep128,
        kron(c1), kron(w_r1_W[:D]),
        kron(w_r2_W), tile8(w_r2_b),
        kron(att1_W[:D]), kron(att1_W[D:]), tile8(att1_b),
        kron(att2_W), tile8(att2_b),
        kron(att3_W),
        kron(jnp.ones((1, D), f32)),
        kron(jnp.ones((1, NR), f32)),
    )
    return out128.reshape(B, D)


# speedup vs baseline: 2.0780x; 1.0135x over previous
import functools

import jax
import jax.numpy as jnp
import numpy as np
from jax import lax
from jax.experimental import pallas as pl
from jax.experimental.pallas import tpu as pltpu
from jax.experimental.pallas import tpu_sc as plsc

B = 16384
L = 50
D = 16
NR = 5
NV = 1000000
NU = 1000000
N = B * L
S = 8
PR = N // S
W128 = S * D

_NC = 2
_NS = 16
_NW = _NC * _NS
_V_PER_W = N // _NW
_U_PER_W = B // _NW
_CHUNK = 2560
_NSTEP = _V_PER_W // _CHUNK


@functools.cache
def _make_sc_gather():
    mesh = plsc.VectorSubcoreMesh(core_axis_name="c", subcore_axis_name="s",
                                  num_cores=_NC, num_subcores=_NS)

    @functools.partial(
        pl.kernel,
        out_type=[
            jax.ShapeDtypeStruct((N, D), jnp.float32),
            jax.ShapeDtypeStruct((B, D), jnp.float32),
        ],
        mesh=mesh,
        compiler_params=pltpu.CompilerParams(use_tc_tiling_on_sc=False),
        scratch_types=[
            pltpu.VMEM((_CHUNK,), jnp.int32),
            pltpu.VMEM((_CHUNK, D), jnp.float32),
            pltpu.VMEM((_U_PER_W,), jnp.int32),
            pltpu.VMEM((_U_PER_W, D), jnp.float32),
            pltpu.SemaphoreType.DMA,
        ],
    )
    def _sc_gather(v2e_hbm, vidx_hbm, u2e_hbm, nodes_hbm,
                   euv_hbm, urep_hbm,
                   idx_v, rows_v, uidx_v, urows_v, sem):
        wid = lax.axis_index("s") * _NC + lax.axis_index("c")
        v2e2 = v2e_hbm
        u2e2 = u2e_hbm

        ubase = pl.multiple_of(wid * _U_PER_W, 8)
        pltpu.sync_copy(nodes_hbm.at[pl.ds(ubase, _U_PER_W)], uidx_v)
        pltpu.async_copy(u2e2.at[uidx_v], urows_v, sem).wait()
        pltpu.sync_copy(urows_v, urep_hbm.at[pl.ds(ubase, _U_PER_W)])

        vbase = wid * _V_PER_W

        def step(i, carry):
            base = pl.multiple_of(vbase + i * _CHUNK, 8)
            pltpu.sync_copy(vidx_hbm.at[pl.ds(base, _CHUNK)], idx_v)
            pltpu.async_copy(v2e2.at[idx_v], rows_v, sem).wait()
            pltpu.sync_copy(rows_v, euv_hbm.at[pl.ds(base, _CHUNK)])
            return carry

        lax.fori_loop(0, _NSTEP, step, 0)

    return _sc_gather


_PBLK = 8192
_NPB = (NV + _PBLK - 1) // _PBLK
_NVP = _NPB * _PBLK


def _sigma(i):
    return ((i >> 13) << 13) | ((i & 1023) << 3) | ((i >> 10) & 7)


def _pack_body(vt_ref, ut_ref, vo_ref, uo_ref):
    f32 = jnp.float32
    dot = functools.partial(jnp.dot, preferred_element_type=f32)
    pb8 = _PBLK // 8

    base = pl.program_id(0) * _PBLK

    def pack(xt):
        col = base + lax.broadcasted_iota(jnp.int32, (D, _PBLK), 1)
        xt = jnp.where(col < NV, xt, 0.0)
        acc = None
        for s in range(8):
            t = jnp.transpose(xt[:, s * pb8:(s + 1) * pb8])
            sh = (lax.broadcasted_iota(jnp.int32, (D, W128), 1) ==
                  lax.broadcasted_iota(jnp.int32, (D, W128), 0) + s * D
                  ).astype(f32)
            y = dot(t, sh)
            acc = y if acc is None else acc + y
        return acc

    vo_ref[...] = pack(vt_ref[...])
    uo_ref[...] = pack(ut_ref[...])


def _pack_tables(v2e_t, u2e_t):
    grid = (_NPB,)
    return pl.pallas_call(
        _pack_body,
        grid=grid,
        in_specs=[pl.BlockSpec((D, _PBLK), lambda i: (0, i)),
                  pl.BlockSpec((D, _PBLK), lambda i: (0, i))],
        out_specs=[pl.BlockSpec((_PBLK // 8, 128), lambda i: (i, 0)),
                   pl.BlockSpec((_PBLK // 8, 128), lambda i: (i, 0))],
        out_shape=[jax.ShapeDtypeStruct((_NVP // 8, 128), jnp.float32),
                   jax.ShapeDtypeStruct((_NVP // 8, 128), jnp.float32)],
        compiler_params=pltpu.CompilerParams(
            dimension_semantics=("parallel",)),
    )(v2e_t, u2e_t)


_ICB = 1024


def _permute_body(uv_ref, r_ref, uvo_ref, ro_ref):
    def perm(x):
        t = jnp.transpose(x)
        return jnp.transpose(t.reshape(_ICB // S, S, L), (0, 2, 1))
    uvo_ref[...] = _sigma(perm(uv_ref[...]))
    ro_ref[...] = perm(r_ref[...])


def _permute_idx(hu_t, hr_t):
    grid = (B // _ICB,)
    spec_in = pl.BlockSpec((L, _ICB), lambda i: (0, i))
    spec_out = pl.BlockSpec((_ICB // S, L, S), lambda i: (i, 0, 0))
    return pl.pallas_call(
        _permute_body,
        grid=grid,
        in_specs=[spec_in, spec_in],
        out_specs=[spec_out, spec_out],
        out_shape=[jax.ShapeDtypeStruct((B // S, L, S), jnp.int32),
                   jax.ShapeDtypeStruct((B // S, L, S), jnp.int32)],
        compiler_params=pltpu.CompilerParams(
            dimension_semantics=("parallel",)),
    )(hu_t, hr_t)


_GB = 16
_BBLK = _GB * S
_MB = _GB * L


def _tc_body(euv_ref, hrp_ref, urep_ref,
             c1p_ref, w1ap_ref, w2p_ref, b2p_ref,
             a1ap_ref, a1bp_ref, ba1p_ref, a2p_ref, ba2p_ref,
             a3p_ref, r8_ref, r5_ref, out_ref):
    f32 = jnp.float32
    dot = functools.partial(jnp.dot, preferred_element_type=f32)

    euv = euv_ref[...]
    hrp = hrp_ref[...].reshape(_MB, S)
    hr40 = dot(hrp.astype(f32), r5_ref[...])
    ohp = (hr40 == (lax.broadcasted_iota(jnp.int32, (_MB, S * NR), 1) % NR
                    ).astype(f32)).astype(f32)

    x1 = jnp.maximum(dot(euv, w1ap_ref[...]) + dot(ohp, c1p_ref[...]), 0.0)
    o = jnp.maximum(dot(x1, w2p_ref[...]) + b2p_ref[...], 0.0)

    etg = (lax.broadcasted_iota(jnp.int32, (_MB, _GB), 0) // L ==
           lax.broadcasted_iota(jnp.int32, (_MB, _GB), 1)).astype(f32)
    eg = (lax.broadcasted_iota(jnp.int32, (_GB, _MB), 0) ==
          lax.broadcasted_iota(jnp.int32, (_GB, _MB), 1) // L).astype(f32)

    u_att = dot(urep_ref[...], a1bp_ref[...]) + ba1p_ref[...]
    u_exp = dot(etg, u_att)

    a1 = jnp.maximum(dot(o, a1ap_ref[...]) + u_exp, 0.0)
    a2 = jnp.maximum(dot(a1, a2p_ref[...]) + ba2p_ref[...], 0.0)
    lg = dot(a2, a3p_ref[...])

    el = jnp.exp(lg)
    den = dot(eg, el)
    dexp = dot(etg, 1.0 / den)
    att = el * dexp
    att128 = dot(att, r8_ref[...])
    out_ref[...] = dot(eg, o * att128)


def _tc_call(euv128, hrp, urep128, c1p, w1ap, w2p, b2p,
             a1ap, a1bp, ba1p, a2p, ba2p, a3p, r8, r5):
    grid = (B // _BBLK,)
    full = lambda shape: pl.BlockSpec(shape, lambda i: (0, 0))
    return pl.pallas_call(
        _tc_body,
        grid=grid,
        in_specs=[
            pl.BlockSpec((_MB, W128), lambda i: (i, 0)),
            pl.BlockSpec((_GB, L, S), lambda i: (i, 0, 0)),
            pl.BlockSpec((_GB, W128), lambda i: (i, 0)),
            full((S * NR, W128)),
            full((W128, W128)), full((W128, W128)), full((1, W128)),
            full((W128, W128)), full((W128, W128)), full((1, W128)),
            full((W128, W128)), full((1, W128)),
            full((W128, S)), full((S, W128)), full((S, S * NR)),
        ],
        out_specs=pl.BlockSpec((_GB, W128), lambda i: (i, 0)),
        out_shape=jax.ShapeDtypeStruct((B // S, W128), jnp.float32),
        compiler_params=pltpu.CompilerParams(
            dimension_semantics=("parallel",)),
    )(euv128, hrp, urep128, c1p, w1ap, w2p, b2p,
      a1ap, a1bp, ba1p, a2p, ba2p, a3p, r8, r5)


def kernel(nodes, history_uv, history_r, v2e, u2e, r2e,
           w_r1_W, w_r1_b, w_r2_W, w_r2_b,
           att1_W, att1_b, att2_W, att2_b, att3_W, att3_b):
    f32 = jnp.float32

    i32 = jnp.int32
    uv3, r3 = _permute_idx(history_uv.astype(i32).T, history_r.astype(i32).T)
    vidx = uv3.reshape(N)
    hrp = r3

    v2e128, u2e128 = _pack_tables(v2e.T, u2e.T)
    euv, urep = _make_sc_gather()(
        v2e128.reshape(_NVP, D), vidx, u2e128.reshape(_NVP, D),
        _sigma(nodes.astype(jnp.int32)))
    euv128 = euv.reshape(PR, W128)
    urep128 = urep.reshape(B // S, W128)

    eye8 = jnp.eye(S, dtype=f32)
    kron = lambda w: jnp.kron(eye8, w.astype(f32))
    tile8 = lambda b: jnp.tile(b.reshape(1, -1), (1, S))

    c1 = r2e @ w_r1_W[D:] + w_r1_b
    out128 = _tc_call(
        euv128, hrp, urep128,
        kron(c1), kron(w_r1_W[:D]),
        kron(w_r2_W), tile8(w_r2_b),
        kron(att1_W[:D]), kron(att1_W[D:]), tile8(att1_b),
        kron(att2_W), tile8(att2_b),
        kron(att3_W),
        kron(jnp.ones((1, D), f32)),
        kron(jnp.ones((1, NR), f32)),
    )
    return out128.reshape(B, D)
